# baseline (speedup 1.0000x reference)
import functools

import jax
import jax.numpy as jnp
from jax import lax
from jax.experimental import pallas as pl
from jax.experimental.pallas import tpu as pltpu
from jax.experimental.pallas import tpu_sc as plsc

MAX_POSN = 8192
D_MODEL = 1024
BATCH = 4
SEQ = 8192
N = BATCH * SEQ

N_TC = 8192
N_SC = N - N_TC

NUM_CORES = 2
NUM_SUBCORES = 16
NW = NUM_CORES * NUM_SUBCORES
BPW = N_SC // NW
CHUNK = 16
NCHUNK = BPW // CHUNK
NBUF = 4
assert NCHUNK % 4 == 0 and NCHUNK >= 8

_mesh = plsc.VectorSubcoreMesh(
    core_axis_name="c", subcore_axis_name="s",
    num_cores=NUM_CORES, num_subcores=NUM_SUBCORES)


@functools.partial(
    pl.kernel,
    out_type=jax.ShapeDtypeStruct((N_SC, D_MODEL), jnp.float32),
    mesh=_mesh,
    scratch_types=[
        pltpu.VMEM((BPW,), jnp.int32),
        [pltpu.VMEM((CHUNK, D_MODEL), jnp.float32) for _ in range(NBUF)],
        [pltpu.SemaphoreType.DMA for _ in range(NBUF)],
        [pltpu.SemaphoreType.DMA for _ in range(NBUF)],
    ],
)
def _gather_sc(posns_hbm, table_hbm, out_hbm, idx_v, bufs, gsems, ssems):
    wid = lax.axis_index("s") * NUM_CORES + lax.axis_index("c")
    base = pl.multiple_of(wid * BPW, BPW)
    pltpu.sync_copy(posns_hbm.at[pl.ds(base, BPW)], idx_v)

    def issue_gather(g, slot):
        off = pl.multiple_of(g * CHUNK, CHUNK)
        pltpu.make_async_copy(
            table_hbm.at[idx_v.at[pl.ds(off, CHUNK)]], bufs[slot],
            gsems[slot]).start()

    def wait_gather(slot):
        pltpu.make_async_copy(
            table_hbm.at[idx_v.at[pl.ds(0, CHUNK)]], bufs[slot],
            gsems[slot]).wait()

    def issue_scatter(g, slot):
        off = pl.multiple_of(g * CHUNK, CHUNK)
        pltpu.make_async_copy(
            bufs[slot], out_hbm.at[pl.ds(base + off, CHUNK)],
            ssems[slot]).start()

    def wait_scatter(slot):
        pltpu.make_async_copy(
            bufs[slot], out_hbm.at[pl.ds(base, CHUNK)], ssems[slot]).wait()

    issue_gather(0, 0)
    issue_gather(1, 1)

    wait_gather(0)
    issue_scatter(0, 0)
    issue_gather(2, 2)

    wait_gather(1)
    issue_scatter(1, 1)
    issue_gather(3, 3)

    def step(c, slot):
        wait_gather(slot)
        issue_scatter(c, slot)
        nxt = (slot + 2) % NBUF
        wait_scatter(nxt)

        @pl.when(c + 2 < NCHUNK)
        def _():
            issue_gather(c + 2, nxt)

    @pl.loop(2, NCHUNK - 2, step=4)
    def _(g):
        step(g, 2)
        step(g + 1, 3)
        step(g + 2, 0)
        step(g + 3, 1)

    wait_gather((NCHUNK - 2) % NBUF)
    issue_scatter(NCHUNK - 2, (NCHUNK - 2) % NBUF)
    wait_scatter((NCHUNK - 4) % NBUF)
    wait_gather((NCHUNK - 1) % NBUF)
    issue_scatter(NCHUNK - 1, (NCHUNK - 1) % NBUF)
    wait_scatter((NCHUNK - 3) % NBUF)
    wait_scatter((NCHUNK - 2) % NBUF)
    wait_scatter((NCHUNK - 1) % NBUF)


R_TC = 8


def _tc_body(idx_ref, *refs):
    del idx_ref
    in_refs = refs[:R_TC]
    out_ref = refs[R_TC]
    for j in range(R_TC):
        out_ref[j, :] = in_refs[j][0, :]


_tc_in_specs = [
    pl.BlockSpec(
        (1, 8, 128),
        functools.partial(
            lambda i, idx_ref, j: (idx_ref[i * R_TC + j], 0, 0), j=j))
    for j in range(R_TC)
]

_gather_tc = pl.pallas_call(
    _tc_body,
    grid_spec=pltpu.PrefetchScalarGridSpec(
        num_scalar_prefetch=1,
        grid=(N_TC // R_TC,),
        in_specs=_tc_in_specs,
        out_specs=pl.BlockSpec((R_TC, 8, 128), lambda i, idx_ref: (i, 0, 0)),
    ),
    out_shape=jax.ShapeDtypeStruct((N_TC, 8, 128), jnp.float32),
)


def kernel(posns, W_pos):
    flat = posns.reshape(N).astype(jnp.int32)
    out_sc = _gather_sc(flat[:N_SC], W_pos)
    table3 = W_pos.reshape(MAX_POSN, 8, 128)
    out_tc = _gather_tc(flat[N_SC:], *([table3] * R_TC))
    out = jnp.concatenate([out_sc, out_tc.reshape(N_TC, D_MODEL)], axis=0)
    return out.reshape(BATCH, SEQ, D_MODEL)


# speedup vs baseline: 6.5819x; 6.5819x over previous
import functools

import jax
import jax.numpy as jnp
from jax import lax
from jax.experimental import pallas as pl
from jax.experimental.pallas import tpu as pltpu
from jax.experimental.pallas import tpu_sc as plsc

MAX_POSN = 8192
D_MODEL = 1024
BATCH = 4
SEQ = 8192
N = BATCH * SEQ

NUM_CORES = 2
NUM_SUBCORES = 16
NW = NUM_CORES * NUM_SUBCORES
BPW = N // NW
CHUNK = 16
NCHUNK = BPW // CHUNK
NBUF = 4

_mesh = plsc.VectorSubcoreMesh(
    core_axis_name="c", subcore_axis_name="s",
    num_cores=NUM_CORES, num_subcores=NUM_SUBCORES)


@functools.partial(
    pl.kernel,
    out_type=jax.ShapeDtypeStruct((N, D_MODEL), jnp.float32),
    mesh=_mesh,
    scratch_types=[
        pltpu.VMEM((BPW,), jnp.int32),
        [pltpu.VMEM((CHUNK, D_MODEL), jnp.float32) for _ in range(NBUF)],
        [pltpu.SemaphoreType.DMA for _ in range(NBUF)],
        [pltpu.SemaphoreType.DMA for _ in range(NBUF)],
    ],
)
def _gather_sc(posns_hbm, table_hbm, out_hbm, idx_v, bufs, gsems, ssems):
    wid = lax.axis_index("s") * NUM_CORES + lax.axis_index("c")
    base = pl.multiple_of(wid * BPW, BPW)
    pltpu.sync_copy(posns_hbm.at[pl.ds(base, BPW)], idx_v)

    def issue_gather(g, slot):
        off = pl.multiple_of(g * CHUNK, CHUNK)
        pltpu.make_async_copy(
            table_hbm.at[idx_v.at[pl.ds(off, CHUNK)]], bufs[slot],
            gsems[slot]).start()

    def wait_gather(slot):
        pltpu.make_async_copy(
            table_hbm.at[idx_v.at[pl.ds(0, CHUNK)]], bufs[slot],
            gsems[slot]).wait()

    def issue_scatter(g, slot):
        off = pl.multiple_of(g * CHUNK, CHUNK)
        pltpu.make_async_copy(
            bufs[slot], out_hbm.at[pl.ds(base + off, CHUNK)],
            ssems[slot]).start()

    def wait_scatter(slot):
        pltpu.make_async_copy(
            bufs[slot], out_hbm.at[pl.ds(base, CHUNK)], ssems[slot]).wait()

    issue_gather(0, 0)
    issue_gather(1, 1)

    wait_gather(0)
    issue_scatter(0, 0)
    issue_gather(2, 2)

    wait_gather(1)
    issue_scatter(1, 1)
    issue_gather(3, 3)

    def step(c, slot):
        nxt = (slot + 2) % NBUF
        wait_scatter(nxt)
        issue_gather(c + 2, nxt)
        wait_gather(slot)
        issue_scatter(c, slot)

    @pl.loop(2, NCHUNK - 2, step=4)
    def _(g):
        step(g, 2)
        step(g + 1, 3)
        step(g + 2, 0)
        step(g + 3, 1)

    wait_scatter((NCHUNK - 4) % NBUF)
    wait_gather((NCHUNK - 2) % NBUF)
    issue_scatter(NCHUNK - 2, (NCHUNK - 2) % NBUF)
    wait_scatter((NCHUNK - 3) % NBUF)
    wait_gather((NCHUNK - 1) % NBUF)
    issue_scatter(NCHUNK - 1, (NCHUNK - 1) % NBUF)
    wait_scatter((NCHUNK - 2) % NBUF)
    wait_scatter((NCHUNK - 1) % NBUF)


def kernel(posns, W_pos):
    flat = posns.reshape(N).astype(jnp.int32)
    out = _gather_sc(flat, W_pos)
    return out.reshape(BATCH, SEQ, D_MODEL)
